# baseline (speedup 1.0000x reference)
import functools

import jax
import jax.numpy as jnp
from jax import lax
from jax.experimental import pallas as pl
from jax.experimental.pallas import tpu as pltpu
from jax.experimental.pallas import tpu_sc as plsc

_NC, _NS = 2, 16
_NW = _NC * _NS
_LP = 208
_NCHUNK = _LP // 16
_QROWS = 32
_NBUF = 4


def _sc_embed(x_pad, pos_flat, B, L, D):
    bpw = B // _NW
    nq = bpw // _QROWS
    flat = L * D
    slab = _LP * D
    mesh = plsc.VectorSubcoreMesh(core_axis_name="c", subcore_axis_name="s")

    @functools.partial(
        pl.kernel,
        mesh=mesh,
        compiler_params=pltpu.CompilerParams(needs_layout_passes=False),
        out_type=jax.ShapeDtypeStruct((B, flat), jnp.float32),
        scratch_types=[
            pltpu.VMEM((2, _QROWS, _LP), jnp.int32),
            pltpu.VMEM((slab,), jnp.float32),
            pltpu.VMEM((slab,), jnp.float32),
            pltpu.VMEM((slab,), jnp.float32),
            pltpu.VMEM((slab,), jnp.float32),
            pltpu.SemaphoreType.DMA,
            pltpu.SemaphoreType.DMA,
            pltpu.SemaphoreType.DMA,
            pltpu.SemaphoreType.DMA,
            pltpu.SemaphoreType.DMA,
        ],
    )
    def k(x_hbm, pos_hbm, out_hbm, idx_v, ob0, ob1, ob2, ob3,
          sem0, sem1, sem2, sem3, isem):
        wid = lax.axis_index("c") * _NS + lax.axis_index("s")
        base = wid * bpw
        bufs = (ob0, ob1, ob2, ob3)
        sems = (sem0, sem1, sem2, sem3)
        for t in range(_NBUF):
            pltpu.sync_copy(pos_hbm, bufs[t].at[pl.ds(0, flat)])
        pltpu.sync_copy(x_hbm.at[pl.ds(base, _QROWS)], idx_v.at[0])

        lane128 = lax.iota(jnp.int32, 16) * D
        pone = jnp.full((16,), 1.0, jnp.float32)
        mone = jnp.full((16,), -1.0, jnp.float32)

        def scatter(buf, p, rq, val):
            for kk in range(_NCHUNK):
                xv = idx_v[p, rq, pl.ds(kk * 16, 16)]
                fidx = lane128 + (kk * 16 * D) + xv
                plsc.addupdate_scatter(buf, [fidx], val)

        def fire(t, b):
            pltpu.async_copy(
                bufs[t].at[pl.ds(0, flat)], out_hbm.at[base + b], sems[t])

        def drain(t, b):
            pltpu.make_async_copy(
                bufs[t].at[pl.ds(0, flat)], out_hbm.at[base + b],
                sems[t]).wait()

        for q in range(nq):
            p = q % 2
            for t in range(_NBUF):
                b = q * _QROWS + t
                if q > 0:
                    drain(t, b - _NBUF)
                    scatter(bufs[t], 1 - p, _QROWS - _NBUF + t, mone)
                scatter(bufs[t], p, t, pone)
                fire(t, b)
            if q + 1 < nq:
                pltpu.async_copy(
                    x_hbm.at[pl.ds(base + (q + 1) * _QROWS, _QROWS)],
                    idx_v.at[1 - p], isem)

            def jiter(j, carry):
                for t in range(_NBUF):
                    rq = j * _NBUF + t
                    b = q * _QROWS + rq
                    drain(t, b - _NBUF)
                    scatter(bufs[t], p, rq - _NBUF, mone)
                    scatter(bufs[t], p, rq, pone)
                    fire(t, b)
                return carry

            lax.fori_loop(1, _QROWS // _NBUF, jiter, 0)
            if q + 1 < nq:
                pltpu.make_async_copy(
                    x_hbm.at[pl.ds(base + (q + 1) * _QROWS, _QROWS)],
                    idx_v.at[1 - p], isem).wait()

        for t in range(_NBUF):
            drain(t, bpw - _NBUF + t)

    return k(x_pad, pos_flat)


def kernel(x, pos_table):
    B, L = x.shape
    D = pos_table.shape[-1]
    x = x.astype(jnp.int32)
    x_pad = jnp.pad(x, ((0, 0), (0, _LP - L)))
    pos_flat = pos_table.reshape(-1)
    out = _sc_embed(x_pad, pos_flat, B, L, D)
    return out.reshape(B, L, D)
